# baseline (speedup 1.0000x reference)
import jax
import jax.numpy as jnp
from jax.experimental import pallas as pl
from jax.experimental.pallas import tpu as pltpu

_L1 = {0: ((1, 1), (3, 0)), 1: ((0, 2), (2, 1))}

_L2 = {
    (0, 0): ((0, 1, 1), (1, 3, 0)),
    (0, 1): ((1, 1, 1), (0, 3, 1)),
    (1, 0): ((1, 0, 1), (0, 2, 1)),
    (1, 1): ((0, 0, 2), (1, 2, 1)),
}

_B = 4


def _body(x_ref, w1_ref, b1_ref, w2_ref, b2_ref, out_ref, hp_ref):
    B = x_ref.shape[0]

    @pl.when(pl.program_id(0) == 0)
    def _zero_scratch():
        hp_ref[...] = jnp.zeros_like(hp_ref)

    xf = x_ref[...].reshape(B * 256, 512)
    a1 = jnp.dot(xf, w1_ref[...], preferred_element_type=jnp.float32)
    a1 = a1.reshape(B, 16, 16, 4096)
    b1 = b1_ref[...]

    for eh in (0, 1):
        for ew in (0, 1):
            acc = None
            for (kh, r0) in _L1[eh]:
                for (kw, c0) in _L1[ew]:
                    t = kh * 4 + kw
                    sl = a1[:, r0:r0 + 14, c0:c0 + 14, t * 256:(t + 1) * 256]
                    acc = sl if acc is None else acc + sl
            hp_ref[:, eh * 2 + ew, 1:15, 1:15, :] = (
                (acc + b1).astype(jnp.bfloat16))

    a2 = jnp.dot(hp_ref[...].reshape(B * 4 * 256, 256), w2_ref[...],
                 preferred_element_type=jnp.float32)
    a2 = a2.reshape(B, 4, 16, 16, 272)
    b2 = b2_ref[...]

    ph = 0
    for fh in (0, 1):
        for gh in (0, 1):
            for fw in (0, 1):
                for gw in (0, 1):
                    acc = None
                    for (eh, kh, v0) in _L2[(fh, gh)]:
                        for (ew, kw, w0) in _L2[(fw, gw)]:
                            t = kh * 4 + kw
                            sl = a2[:, eh * 2 + ew, v0:v0 + 14,
                                    w0:w0 + 14, t * 17:(t + 1) * 17]
                            acc = sl if acc is None else acc + sl
                    out_ref[:, ph] = acc + b2
                    ph += 1


def _call(xp, w1t, b1r, w2t, b2r):
    n = xp.shape[0]
    grid = (n // _B,)
    return pl.pallas_call(
        _body,
        grid=grid,
        in_specs=[
            pl.BlockSpec((_B, 16, 16, 512), lambda i: (i, 0, 0, 0)),
            pl.BlockSpec((512, 4096), lambda i: (0, 0)),
            pl.BlockSpec((1, 256), lambda i: (0, 0)),
            pl.BlockSpec((256, 272), lambda i: (0, 0)),
            pl.BlockSpec((1, 17), lambda i: (0, 0)),
        ],
        out_specs=pl.BlockSpec((_B, 16, 14, 14, 17),
                               lambda i: (i, 0, 0, 0, 0)),
        out_shape=jax.ShapeDtypeStruct((n, 16, 14, 14, 17), jnp.float32),
        scratch_shapes=[pltpu.VMEM((_B, 4, 16, 16, 256), jnp.bfloat16)],
        compiler_params=pltpu.CompilerParams(
            dimension_semantics=("arbitrary",),
        ),
    )(xp, w1t, b1r, w2t, b2r)


def kernel(x, W1, b1, W2, b2):
    n = x.shape[0]
    xp = jnp.pad(x.transpose(0, 2, 3, 1),
                 ((0, 0), (1, 1), (1, 1), (0, 0))).astype(jnp.bfloat16)
    w1t = W1.transpose(0, 2, 3, 1).reshape(512, 4096).astype(jnp.bfloat16)
    w2t = W2.transpose(0, 2, 3, 1).reshape(256, 272).astype(jnp.bfloat16)
    b1r, b2r = b1.reshape(1, 256), b2.reshape(1, 17)
    chunks = []
    nc = 8
    cs = n // nc
    for k in range(nc):
        o = _call(xp[k * cs:(k + 1) * cs], w1t, b1r, w2t, b2r)
        o = o.reshape(cs, 2, 2, 2, 2, 14, 14, 17)
        o = o.transpose(0, 7, 5, 2, 1, 6, 4, 3).reshape(cs, 17, 56, 56)
        chunks.append(o)
    return jnp.concatenate(chunks, axis=0)


# speedup vs baseline: 1.8243x; 1.8243x over previous
import jax
import jax.numpy as jnp
from jax.experimental import pallas as pl
from jax.experimental.pallas import tpu as pltpu

_L1 = {0: ((1, 1), (3, 0)), 1: ((0, 2), (2, 1))}

_L2 = {
    (0, 0): ((0, 1, 1), (1, 3, 0)),
    (0, 1): ((1, 1, 1), (0, 3, 1)),
    (1, 0): ((1, 0, 1), (0, 2, 1)),
    (1, 1): ((0, 0, 2), (1, 2, 1)),
}

_B = 4


def _body(x_ref, w1_ref, b1_ref, w2_ref, b2_ref, out_ref, hp_ref):
    B = x_ref.shape[0]

    @pl.when(pl.program_id(0) == 0)
    def _zero_scratch():
        hp_ref[...] = jnp.zeros_like(hp_ref)

    xf = x_ref[...].reshape(B * 256, 512)
    a1 = jnp.dot(xf, w1_ref[...], preferred_element_type=jnp.float32)
    a1 = a1.reshape(B, 16, 16, 4096)
    b1 = b1_ref[...]

    a2s = []
    for eh in (0, 1):
        for ew in (0, 1):
            acc = None
            for (kh, r0) in _L1[eh]:
                for (kw, c0) in _L1[ew]:
                    t = kh * 4 + kw
                    sl = a1[:, r0:r0 + 14, c0:c0 + 14, t * 256:(t + 1) * 256]
                    acc = sl if acc is None else acc + sl
            e = eh * 2 + ew
            hp_ref[:, e, 1:15, 1:15, :] = (acc + b1).astype(jnp.bfloat16)
            a2e = jnp.dot(hp_ref[:, e].reshape(B * 256, 256), w2_ref[...],
                          preferred_element_type=jnp.float32)
            a2s.append(a2e.reshape(B, 16, 16, 272))
    b2 = b2_ref[...]

    ph = 0
    for fh in (0, 1):
        for gh in (0, 1):
            for fw in (0, 1):
                for gw in (0, 1):
                    acc = None
                    for (eh, kh, v0) in _L2[(fh, gh)]:
                        for (ew, kw, w0) in _L2[(fw, gw)]:
                            t = kh * 4 + kw
                            sl = a2s[eh * 2 + ew][:, v0:v0 + 14,
                                                  w0:w0 + 14,
                                                  t * 17:(t + 1) * 17]
                            acc = sl if acc is None else acc + sl
                    out_ref[:, ph] = acc + b2
                    ph += 1


def _call(xp, w1t, b1r, w2t, b2r):
    n = xp.shape[0]
    grid = (n // _B,)
    return pl.pallas_call(
        _body,
        grid=grid,
        in_specs=[
            pl.BlockSpec((_B, 16, 16, 512), lambda i: (i, 0, 0, 0)),
            pl.BlockSpec((512, 4096), lambda i: (0, 0)),
            pl.BlockSpec((1, 256), lambda i: (0, 0)),
            pl.BlockSpec((256, 272), lambda i: (0, 0)),
            pl.BlockSpec((1, 17), lambda i: (0, 0)),
        ],
        out_specs=pl.BlockSpec((_B, 16, 14, 14, 17),
                               lambda i: (i, 0, 0, 0, 0)),
        out_shape=jax.ShapeDtypeStruct((n, 16, 14, 14, 17), jnp.float32),
        scratch_shapes=[pltpu.VMEM((_B, 4, 16, 16, 256), jnp.bfloat16)],
        compiler_params=pltpu.CompilerParams(
            dimension_semantics=("arbitrary",),
        ),
    )(xp, w1t, b1r, w2t, b2r)


def kernel(x, W1, b1, W2, b2):
    n = x.shape[0]
    w1t = W1.transpose(0, 2, 3, 1).reshape(512, 4096).astype(jnp.bfloat16)
    w2t = W2.transpose(0, 2, 3, 1).reshape(256, 272).astype(jnp.bfloat16)
    b1r, b2r = b1.reshape(1, 256), b2.reshape(1, 17)
    chunks = []
    nc = 4
    cs = n // nc
    for k in range(nc):
        xp = jnp.pad(x[k * cs:(k + 1) * cs].transpose(0, 2, 3, 1),
                     ((0, 0), (1, 1), (1, 1), (0, 0))).astype(jnp.bfloat16)
        o = _call(xp, w1t, b1r, w2t, b2r)
        o = o.reshape(cs, 2, 2, 2, 2, 14, 14, 17)
        o = o.transpose(0, 7, 5, 2, 1, 6, 4, 3).reshape(cs, 17, 56, 56)
        chunks.append(o)
    return jnp.concatenate(chunks, axis=0)


# speedup vs baseline: 1.8263x; 1.0011x over previous
import jax
import jax.numpy as jnp
from jax.experimental import pallas as pl
from jax.experimental.pallas import tpu as pltpu

_L1 = {0: ((1, 1), (3, 0)), 1: ((0, 2), (2, 1))}

_L2 = {
    (0, 0): ((0, 1, 1), (1, 3, 0)),
    (0, 1): ((1, 1, 1), (0, 3, 1)),
    (1, 0): ((1, 0, 1), (0, 2, 1)),
    (1, 1): ((0, 0, 2), (1, 2, 1)),
}

_B = 4


def _body(x_ref, w1_ref, b1_ref, w2_ref, b2_ref, out_ref, hp_ref):
    B = x_ref.shape[0]

    @pl.when(pl.program_id(0) == 0)
    def _zero_scratch():
        hp_ref[...] = jnp.zeros_like(hp_ref)

    xf = x_ref[...].reshape(B * 256, 512)
    a1 = jnp.dot(xf, w1_ref[...], preferred_element_type=jnp.float32)
    a1 = a1.reshape(B, 16, 16, 4096)
    b1 = b1_ref[...]

    for eh in (0, 1):
        for ew in (0, 1):
            acc = None
            for (kh, r0) in _L1[eh]:
                for (kw, c0) in _L1[ew]:
                    t = kh * 4 + kw
                    sl = a1[:, r0:r0 + 14, c0:c0 + 14, t * 256:(t + 1) * 256]
                    acc = sl if acc is None else acc + sl
            hp_ref[:, eh * 2 + ew, 1:15, 1:15, :] = (
                (acc + b1).astype(jnp.bfloat16))

    a2 = jnp.dot(hp_ref[...].reshape(B * 4 * 256, 256), w2_ref[...],
                 preferred_element_type=jnp.float32)
    a2 = a2.reshape(B, 4, 16, 16, 272)
    b2 = b2_ref[...]

    ph = 0
    for fh in (0, 1):
        for gh in (0, 1):
            for fw in (0, 1):
                for gw in (0, 1):
                    acc = None
                    for (eh, kh, v0) in _L2[(fh, gh)]:
                        for (ew, kw, w0) in _L2[(fw, gw)]:
                            t = kh * 4 + kw
                            sl = a2[:, eh * 2 + ew, v0:v0 + 14,
                                    w0:w0 + 14, t * 17:(t + 1) * 17]
                            acc = sl if acc is None else acc + sl
                    out_ref[:, ph] = acc + b2
                    ph += 1


def _call(xp, w1t, b1r, w2t, b2r):
    n = xp.shape[0]
    grid = (n // _B,)
    return pl.pallas_call(
        _body,
        grid=grid,
        in_specs=[
            pl.BlockSpec((_B, 16, 16, 512), lambda i: (i, 0, 0, 0)),
            pl.BlockSpec((512, 4096), lambda i: (0, 0)),
            pl.BlockSpec((1, 256), lambda i: (0, 0)),
            pl.BlockSpec((256, 272), lambda i: (0, 0)),
            pl.BlockSpec((1, 17), lambda i: (0, 0)),
        ],
        out_specs=pl.BlockSpec((_B, 16, 14, 14, 17),
                               lambda i: (i, 0, 0, 0, 0)),
        out_shape=jax.ShapeDtypeStruct((n, 16, 14, 14, 17), jnp.float32),
        scratch_shapes=[pltpu.VMEM((_B, 4, 16, 16, 256), jnp.bfloat16)],
        compiler_params=pltpu.CompilerParams(
            dimension_semantics=("arbitrary",),
        ),
    )(xp, w1t, b1r, w2t, b2r)


def kernel(x, W1, b1, W2, b2):
    n = x.shape[0]
    w1t = W1.transpose(0, 2, 3, 1).reshape(512, 4096).astype(jnp.bfloat16)
    w2t = W2.transpose(0, 2, 3, 1).reshape(256, 272).astype(jnp.bfloat16)
    b1r, b2r = b1.reshape(1, 256), b2.reshape(1, 17)
    chunks = []
    nc = 4
    cs = n // nc
    for k in range(nc):
        xp = jnp.pad(x[k * cs:(k + 1) * cs].transpose(0, 2, 3, 1),
                     ((0, 0), (1, 1), (1, 1), (0, 0))).astype(jnp.bfloat16)
        o = _call(xp, w1t, b1r, w2t, b2r)
        o = o.reshape(cs, 2, 2, 2, 2, 14, 14, 17)
        o = o.transpose(0, 7, 5, 2, 1, 6, 4, 3).reshape(cs, 17, 56, 56)
        chunks.append(o)
    return jnp.concatenate(chunks, axis=0)


# speedup vs baseline: 2.0077x; 1.0993x over previous
import jax
import jax.numpy as jnp
from jax.experimental import pallas as pl
from jax.experimental.pallas import tpu as pltpu

_L1 = {0: ((1, 1), (3, 0)), 1: ((0, 2), (2, 1))}

_L2 = {
    (0, 0): ((0, 1, 1), (1, 3, 0)),
    (0, 1): ((1, 1, 1), (0, 3, 1)),
    (1, 0): ((1, 0, 1), (0, 2, 1)),
    (1, 1): ((0, 0, 2), (1, 2, 1)),
}

_B = 4


def _body(x_ref, w1_ref, b1_ref, w2_ref, b2_ref, out_ref, hp_ref):
    B = x_ref.shape[0]

    @pl.when(pl.program_id(0) == 0)
    def _zero_scratch():
        hp_ref[...] = jnp.zeros_like(hp_ref)

    xf = x_ref[...].reshape(B * 256, 512)
    a1 = jnp.dot(xf, w1_ref[...], preferred_element_type=jnp.float32)
    a1 = a1.reshape(B, 16, 16, 4096)
    b1 = b1_ref[...]

    for eh in (0, 1):
        for ew in (0, 1):
            acc = None
            for (kh, r0) in _L1[eh]:
                for (kw, c0) in _L1[ew]:
                    t = kh * 4 + kw
                    sl = a1[:, r0:r0 + 14, c0:c0 + 14, t * 256:(t + 1) * 256]
                    acc = sl if acc is None else acc + sl
            hp_ref[:, eh * 2 + ew, 1:15, 1:15, :] = (
                (acc + b1).astype(jnp.bfloat16))

    a2 = jnp.dot(hp_ref[...].reshape(B * 4 * 256, 256), w2_ref[...],
                 preferred_element_type=jnp.float32)
    a2 = a2.reshape(B, 4, 16, 16, 272)
    b2 = b2_ref[...]

    ph = 0
    for fh in (0, 1):
        for gh in (0, 1):
            for fw in (0, 1):
                for gw in (0, 1):
                    acc = None
                    for (eh, kh, v0) in _L2[(fh, gh)]:
                        for (ew, kw, w0) in _L2[(fw, gw)]:
                            t = kh * 4 + kw
                            sl = a2[:, eh * 2 + ew, v0:v0 + 14,
                                    w0:w0 + 14, t * 17:(t + 1) * 17]
                            acc = sl if acc is None else acc + sl
                    out_ref[:, ph] = acc + b2
                    ph += 1


def _call(xp, w1t, b1r, w2t, b2r):
    n = xp.shape[0]
    grid = (n // _B,)
    return pl.pallas_call(
        _body,
        grid=grid,
        in_specs=[
            pl.BlockSpec((_B, 16, 16, 512), lambda i: (i, 0, 0, 0)),
            pl.BlockSpec((512, 4096), lambda i: (0, 0)),
            pl.BlockSpec((1, 256), lambda i: (0, 0)),
            pl.BlockSpec((256, 272), lambda i: (0, 0)),
            pl.BlockSpec((1, 17), lambda i: (0, 0)),
        ],
        out_specs=pl.BlockSpec((_B, 16, 14, 14, 17),
                               lambda i: (i, 0, 0, 0, 0)),
        out_shape=jax.ShapeDtypeStruct((n, 16, 14, 14, 17), jnp.float32),
        scratch_shapes=[pltpu.VMEM((_B, 4, 16, 16, 256), jnp.bfloat16)],
        compiler_params=pltpu.CompilerParams(
            dimension_semantics=("arbitrary",),
        ),
    )(xp, w1t, b1r, w2t, b2r)


def kernel(x, W1, b1, W2, b2):
    n = x.shape[0]
    w1t = W1.transpose(0, 2, 3, 1).reshape(512, 4096).astype(jnp.bfloat16)
    w2t = W2.transpose(0, 2, 3, 1).reshape(256, 272).astype(jnp.bfloat16)
    b1r, b2r = b1.reshape(1, 256), b2.reshape(1, 17)
    xp = jnp.pad(x.transpose(0, 2, 3, 1),
                 ((0, 0), (1, 1), (1, 1), (0, 0))).astype(jnp.bfloat16)
    chunks = []
    nc = 4
    cs = n // nc
    for k in range(nc):
        o = _call(xp[k * cs:(k + 1) * cs], w1t, b1r, w2t, b2r)
        o = o.reshape(cs, 2, 2, 2, 2, 14, 14, 17)
        o = o.transpose(0, 7, 5, 2, 1, 6, 4, 3).reshape(cs, 17, 56, 56)
        chunks.append(o)
    return jnp.concatenate(chunks, axis=0)
